# Initial kernel scaffold; baseline (speedup 1.0000x reference)
import functools

import jax
import jax.numpy as jnp
from jax import lax
from jax.experimental import pallas as pl
from jax.experimental.pallas import tpu as pltpu
from jax.experimental.pallas import tpu_sc as plsc

EMB = 32
NUM_CORES = 2
NUM_SUBCORES = 16
NW = NUM_CORES * NUM_SUBCORES
CHUNK = 1280


@functools.partial(jax.jit, static_argnames=("n_total",))
def _sc_gather(ids_flat, table, n_total):
    b_per_w = n_total // NW
    n_chunks = b_per_w // CHUNK
    mesh = plsc.VectorSubcoreMesh(core_axis_name="c", subcore_axis_name="s")

    @functools.partial(
        pl.kernel,
        mesh=mesh,
        out_type=jax.ShapeDtypeStruct((n_total, EMB), jnp.float32),
        scratch_types=[
            pltpu.VMEM((CHUNK,), jnp.int32),
            pltpu.VMEM((CHUNK, EMB), jnp.float32),
            pltpu.SemaphoreType.DMA,
        ],
    )
    def k(ids_hbm, table_hbm, out_hbm, idx_v, rows_v, sem):
        wid = lax.axis_index("s") * NUM_CORES + lax.axis_index("c")
        base = wid * b_per_w

        def body(i, _):
            off = base + i * CHUNK
            pltpu.sync_copy(ids_hbm.at[pl.ds(off, CHUNK)], idx_v)
            pltpu.async_copy(table_hbm.at[idx_v], rows_v, sem).wait()
            pltpu.sync_copy(rows_v, out_hbm.at[pl.ds(off, CHUNK)])
            return 0

        lax.fori_loop(0, n_chunks, body, 0)

    return k(ids_flat, table)


def kernel(enum_ids, emb_table):
    ids_flat = enum_ids.reshape(-1).astype(jnp.int32)
    out = _sc_gather(ids_flat, emb_table, ids_flat.shape[0])
    return out.reshape(enum_ids.shape + (EMB,))


# speedup vs baseline: 1.0990x; 1.0990x over previous
import functools

import jax
import jax.numpy as jnp
from jax import lax
from jax.experimental import pallas as pl
from jax.experimental.pallas import tpu as pltpu
from jax.experimental.pallas import tpu_sc as plsc

EMB = 32
NUM_CORES = 2
NUM_SUBCORES = 16
NW = NUM_CORES * NUM_SUBCORES
CHUNK = 1280


@functools.partial(jax.jit, static_argnames=("n_total",))
def _sc_gather(ids_flat, table, n_total):
    b_per_w = n_total // NW
    n_chunks = b_per_w // CHUNK
    mesh = plsc.VectorSubcoreMesh(core_axis_name="c", subcore_axis_name="s")

    @functools.partial(
        pl.kernel,
        mesh=mesh,
        out_type=jax.ShapeDtypeStruct((n_total, EMB), jnp.float32),
        scratch_types=[
            pltpu.VMEM((CHUNK,), jnp.int32),
            pltpu.VMEM((CHUNK, EMB), jnp.float32),
            pltpu.SemaphoreType.DMA,
        ],
        compiler_params=pltpu.CompilerParams(use_tc_tiling_on_sc=False),
    )
    def k(ids_hbm, table_hbm, out_hbm, idx_v, rows_v, sem):
        wid = lax.axis_index("s") * NUM_CORES + lax.axis_index("c")
        base = wid * b_per_w

        def body(i, _):
            off = base + i * CHUNK
            pltpu.sync_copy(ids_hbm.at[pl.ds(off, CHUNK)], idx_v)
            pltpu.async_copy(table_hbm.at[idx_v], rows_v, sem).wait()
            pltpu.sync_copy(rows_v, out_hbm.at[pl.ds(off, CHUNK)])
            return 0

        lax.fori_loop(0, n_chunks, body, 0)

    return k(ids_flat, table)


def kernel(enum_ids, emb_table):
    ids_flat = enum_ids.reshape(-1).astype(jnp.int32)
    out = _sc_gather(ids_flat, emb_table, ids_flat.shape[0])
    return out.reshape(enum_ids.shape + (EMB,))


# speedup vs baseline: 1.3050x; 1.1874x over previous
import functools

import jax
import jax.numpy as jnp
from jax import lax
from jax.experimental import pallas as pl
from jax.experimental.pallas import tpu as pltpu
from jax.experimental.pallas import tpu_sc as plsc

B, L, EMB = 16384, 50, 32
N_TOTAL = B * L
PAD = 128
NUM_CORES = 2
NW = 32
CHUNK = 1280
B_PER_W = N_TOTAL // NW
N_CHUNKS = B_PER_W // CHUNK


@jax.jit
def _sc_gather(ids_flat, table):
    mesh = plsc.VectorSubcoreMesh(core_axis_name="c", subcore_axis_name="s")

    @functools.partial(
        pl.kernel,
        mesh=mesh,
        out_type=jax.ShapeDtypeStruct((N_TOTAL, PAD), jnp.float32),
        scratch_types=[
            pltpu.VMEM((CHUNK,), jnp.int32),
            pltpu.VMEM((CHUNK, EMB), jnp.float32),
            pltpu.SemaphoreType.DMA,
        ],
        compiler_params=pltpu.CompilerParams(use_tc_tiling_on_sc=False),
    )
    def k(ids_hbm, table_hbm, out_hbm, idx_v, rows_v, sem):
        wid = lax.axis_index("s") * NUM_CORES + lax.axis_index("c")
        base = wid * B_PER_W

        def body(i, _):
            off = base + i * CHUNK
            pltpu.sync_copy(ids_hbm.at[pl.ds(off, CHUNK)], idx_v)
            pltpu.async_copy(table_hbm.at[idx_v], rows_v, sem).wait()
            pltpu.sync_copy(rows_v, out_hbm.at[pl.ds(off, CHUNK), pl.ds(0, EMB)])
            return 0

        lax.fori_loop(0, N_CHUNKS, body, 0)

    return k(ids_flat, table)


def kernel(enum_ids, emb_table):
    ids_flat = enum_ids.reshape(-1).astype(jnp.int32)
    y = _sc_gather(ids_flat, emb_table)
    return y[:, :EMB].reshape(B, L, EMB)


# speedup vs baseline: 1.3207x; 1.0121x over previous
import functools

import jax
import jax.numpy as jnp
from jax import lax
from jax.experimental import pallas as pl
from jax.experimental.pallas import tpu as pltpu
from jax.experimental.pallas import tpu_sc as plsc

B, L, EMB = 16384, 50, 32
N_TOTAL = B * L
PAD = 128
NUM_CORES = 2
NW = 32
CHUNK = 1280
B_PER_W = N_TOTAL // NW
N_CHUNKS = B_PER_W // CHUNK


@jax.jit
def _sc_gather(ids_flat, table):
    mesh = plsc.VectorSubcoreMesh(core_axis_name="c", subcore_axis_name="s")

    @functools.partial(
        pl.kernel,
        mesh=mesh,
        out_type=jax.ShapeDtypeStruct((N_TOTAL, PAD), jnp.float32),
        scratch_types=[
            pltpu.VMEM((CHUNK,), jnp.int32),
            pltpu.VMEM((CHUNK,), jnp.int32),
            pltpu.VMEM((CHUNK, EMB), jnp.float32),
            pltpu.VMEM((CHUNK, EMB), jnp.float32),
            pltpu.SemaphoreType.DMA,
            pltpu.SemaphoreType.DMA,
            pltpu.SemaphoreType.DMA,
            pltpu.SemaphoreType.DMA,
            pltpu.SemaphoreType.DMA,
        ],
        compiler_params=pltpu.CompilerParams(use_tc_tiling_on_sc=False),
    )
    def k(ids_hbm, table_hbm, out_hbm, idx0, idx1, rows0, rows1,
          si0, si1, ss0, ss1, sg):
        idx = (idx0, idx1)
        rows = (rows0, rows1)
        si = (si0, si1)
        ss = (ss0, ss1)

        wid = lax.axis_index("s") * NUM_CORES + lax.axis_index("c")
        base = wid * B_PER_W

        def ids_slice(c):
            return ids_hbm.at[pl.ds(base + c * CHUNK, CHUNK)]

        def out_slice(c):
            return out_hbm.at[pl.ds(base + c * CHUNK, CHUNK), pl.ds(0, EMB)]

        pltpu.async_copy(ids_slice(0), idx[0], si[0])
        pltpu.async_copy(ids_slice(1), idx[1], si[1])

        def pair_body(g, _):
            for b in range(2):
                c = g * 2 + b
                pltpu.make_async_copy(ids_slice(c), idx[b], si[b]).wait()
                @pl.when(c >= 2)
                def _():
                    pltpu.make_async_copy(rows[b], out_slice(c), ss[b]).wait()
                pltpu.async_copy(table_hbm.at[idx[b]], rows[b], sg).wait()
                pltpu.async_copy(rows[b], out_slice(c), ss[b])
                @pl.when(c <= N_CHUNKS - 3)
                def _():
                    pltpu.async_copy(ids_slice(c + 2), idx[b], si[b])
            return 0

        lax.fori_loop(0, N_CHUNKS // 2, pair_body, 0)

        pltpu.make_async_copy(rows[0], out_slice(N_CHUNKS - 2), ss[0]).wait()
        pltpu.make_async_copy(rows[1], out_slice(N_CHUNKS - 1), ss[1]).wait()

    return k(ids_flat, table)


def kernel(enum_ids, emb_table):
    ids_flat = enum_ids.reshape(-1).astype(jnp.int32)
    y = _sc_gather(ids_flat, emb_table)
    return y[:, :EMB].reshape(B, L, EMB)
